# baseline (speedup 1.0000x reference)
import functools

import jax
import jax.numpy as jnp
from jax import lax
from jax.experimental import pallas as pl
from jax.experimental.pallas import tpu as pltpu
from jax.experimental.pallas import tpu_sc as plsc

B = 1024
R = 256
S = 8192
F = 2048

SC_BATCHES = 1
SB = 512
FW = 128
NG = FW // 16
NACT = F * SC_BATCHES // FW


def _tc_cumsum_kernel(x_ref, o_ref, carry_ref, *, blk, grp):
    sblk = pl.program_id(1)

    @pl.when(sblk == 0)
    def _():
        carry_ref[...] = jnp.zeros_like(carry_ref)

    tri = jnp.tril(jnp.ones((grp, grp), dtype=jnp.float32)).astype(jnp.bfloat16)
    carry = carry_ref[...]
    for g in range(blk // grp):
        xg = x_ref[0, g * grp : (g + 1) * grp, :]
        local = jax.lax.dot(
            tri, xg.astype(jnp.bfloat16), preferred_element_type=jnp.float32
        )
        out = local + carry
        o_ref[0, g * grp : (g + 1) * grp, :] = out
        carry = out[grp - 1 :, :]
    carry_ref[...] = carry


def _sc_cumsum(x_hbm, out_hbm, buf):
    c = lax.axis_index("c")
    sub = lax.axis_index("s")
    wid = sub * 2 + c
    nstrips = F // FW
    b = 4 - SC_BATCHES + wid // nstrips
    ob = wid // nstrips
    f0 = (wid % nstrips) * FW

    @pl.when(wid < NACT)
    def _():
        def chunk_body(cs, carry):
            row0 = cs * SB
            pltpu.sync_copy(x_hbm.at[b, pl.ds(row0, SB), pl.ds(f0, FW)], buf)

            def row_body(r, carry):
                new = []
                for g in range(NG):
                    v = buf[r, pl.ds(g * 16, 16)] + carry[g]
                    buf[r, pl.ds(g * 16, 16)] = v
                    new.append(v)
                return tuple(new)

            carry = lax.fori_loop(0, SB, row_body, carry)
            pltpu.sync_copy(buf, out_hbm.at[ob, pl.ds(row0, SB), pl.ds(f0, FW)])
            return carry

        init = tuple(jnp.zeros((16,), jnp.float32) for _ in range(NG))
        lax.fori_loop(0, S // SB, chunk_body, init)


def kernel(x):
    tc_batches = x.shape[0] - SC_BATCHES
    tc = pl.pallas_call(
        functools.partial(_tc_cumsum_kernel, blk=B, grp=R),
        grid=(tc_batches, S // B),
        in_specs=[pl.BlockSpec((1, B, F), lambda b, s: (b, s, 0))],
        out_specs=pl.BlockSpec((1, B, F), lambda b, s: (b, s, 0)),
        out_shape=jax.ShapeDtypeStruct((tc_batches, S, F), x.dtype),
        scratch_shapes=[pltpu.VMEM((1, F), jnp.float32)],
        compiler_params=pltpu.CompilerParams(
            dimension_semantics=("parallel", "arbitrary"),
        ),
    )
    sc = pl.kernel(
        _sc_cumsum,
        mesh=plsc.VectorSubcoreMesh(core_axis_name="c", subcore_axis_name="s"),
        out_type=jax.ShapeDtypeStruct((SC_BATCHES, S, F), x.dtype),
        scratch_types=[pltpu.VMEM((SB, FW), jnp.float32)],
    )
    return jnp.concatenate([tc(x), sc(x)], axis=0)


# speedup vs baseline: 2.1733x; 2.1733x over previous
import functools

import jax
import jax.numpy as jnp
from jax.experimental import pallas as pl
from jax.experimental.pallas import tpu as pltpu

B = 1024
R = 256
S = 8192
F = 2048


def _cumsum_kernel(x_ref, o_ref, carry_ref, *, blk, grp):
    s = pl.program_id(1)

    @pl.when(s == 0)
    def _():
        carry_ref[...] = jnp.zeros_like(carry_ref)

    tri = jnp.tril(jnp.ones((grp, grp), dtype=jnp.float32)).astype(jnp.bfloat16)
    carry = carry_ref[...]
    for g in range(blk // grp):
        xg = x_ref[0, g * grp : (g + 1) * grp, :]
        local = jax.lax.dot(
            tri, xg.astype(jnp.bfloat16), preferred_element_type=jnp.float32
        )
        out = local + carry
        o_ref[0, g * grp : (g + 1) * grp, :] = out
        carry = out[grp - 1 :, :]
    carry_ref[...] = carry


def kernel(x):
    batch = x.shape[0]
    grid = (batch, S // B)
    f = pl.pallas_call(
        functools.partial(_cumsum_kernel, blk=B, grp=R),
        grid=grid,
        in_specs=[pl.BlockSpec((1, B, F), lambda b, s: (b, s, 0))],
        out_specs=pl.BlockSpec((1, B, F), lambda b, s: (b, s, 0)),
        out_shape=jax.ShapeDtypeStruct(x.shape, x.dtype),
        scratch_shapes=[pltpu.VMEM((1, F), jnp.float32)],
        compiler_params=pltpu.CompilerParams(
            dimension_semantics=("parallel", "arbitrary"),
        ),
    )
    return f(x)
